# baseline (speedup 1.0000x reference)
import jax
import jax.numpy as jnp
from jax import lax
from jax.experimental import pallas as pl
from jax.experimental.pallas import tpu as pltpu
from jax.experimental.pallas import tpu_sc as plsc

N = 10000
E = 320000
D_IN = 128
D_H = 256
D_OUT = 128

NC = 2
NS = 16
NW = NC * NS
EPW = E // NW
CHUNK = 80
NCHUNK = EPW // CHUNK
NBUF = 4
NMETA = 8
GDEPTH = NBUF - 1
GROUPS = NCHUNK // NMETA
TAIL = NCHUNK - GROUPS * NMETA
ROWS_A = 624
ROWS_B = 632
SPLIT_TILE = 14


def _sc_agg_body(x_hbm, src_hbm, dst_hbm, w_hbm, out_hbm, *scr):
    msrc = scr[0:NMETA]
    mdst = scr[NMETA:2 * NMETA]
    mw = scr[2 * NMETA:3 * NMETA]
    rows = scr[3 * NMETA:3 * NMETA + NBUF]
    agg_sh = scr[3 * NMETA + NBUF]
    sems = scr[3 * NMETA + NBUF + 1:]
    gsem = sems[0:NBUF]
    ssem = sems[NBUF:2 * NBUF]
    msem = sems[2 * NBUF:2 * NBUF + NMETA]
    c = lax.axis_index("c")
    s = lax.axis_index("s")
    wid = c * NS + s
    ebase = wid * EPW

    def m_descs(k, t):
        sl = pl.ds(ebase + k * CHUNK, CHUNK)
        return (pltpu.make_async_copy(src_hbm.at[sl], msrc[t], msem[t]),
                pltpu.make_async_copy(dst_hbm.at[sl], mdst[t], msem[t]),
                pltpu.make_async_copy(w_hbm.at[sl], mw[t], msem[t]))

    def m_start(k, t):
        for d in m_descs(k, t):
            d.start()

    def m_wait(k, t):
        for d in m_descs(k, t):
            d.wait()

    def g_desc(b, t):
        return pltpu.make_async_copy(x_hbm.at[msrc[t]], rows[b], gsem[b])

    def s_desc(b, t):
        return pltpu.make_async_copy(rows[b], agg_sh.at[mdst[t]], ssem[b])

    def scale(b, t):
        @plsc.parallel_loop(0, CHUNK, step=1, unroll=16)
        def _scale(e):
            wsp = plsc.load_gather(mw[t], [jnp.full((16,), e, jnp.int32)])
            for jj in range(D_IN // 16):
                sl = pl.ds(jj * 16, 16)
                rows[b][e, sl] = rows[b][e, sl] * wsp

    zbuf = rows[NBUF - 1]
    for t in range(NMETA - 1):
        m_start(t, t)
    for t in range(GDEPTH):
        m_wait(t, t)
        g_desc(t, t).start()

    zv = jnp.zeros((16,), jnp.float32)

    def zero_body(i, carry):
        for j in range(D_IN // 16):
            zbuf[i, pl.ds(j * 16, 16)] = zv
        return carry

    lax.fori_loop(0, CHUNK, zero_body, 0)

    base = jnp.where(s < SPLIT_TILE, s * ROWS_A, s * ROWS_B - 112)

    def _slice_copies(nrows, copy_fn):
        nfull = nrows // CHUNK
        rem = nrows - nfull * CHUNK
        for kk in range(nfull):
            copy_fn(pl.ds(base + kk * CHUNK, CHUNK), CHUNK)
        if rem:
            copy_fn(pl.ds(base + nfull * CHUNK, rem), rem)

    def _zero_copy(dst_sl, nr):
        pltpu.sync_copy(zbuf.at[pl.ds(0, nr)], agg_sh.at[dst_sl])

    @pl.when(s < SPLIT_TILE)
    def _zero_slice_a():
        _slice_copies(ROWS_A, _zero_copy)

    @pl.when(s >= SPLIT_TILE)
    def _zero_slice_b():
        _slice_copies(ROWS_B, _zero_copy)

    plsc.subcore_barrier()

    def group_body(j, carry):
        for i in range(NMETA):
            k = j * NMETA + i
            b = i % NBUF
            t = i
            bg = (i + GDEPTH) % NBUF
            tg = (i + GDEPTH) % NMETA
            tm = (i + NMETA - 1) % NMETA
            g_desc(b, t).wait()
            scale(b, t)

            if i == 0:
                @pl.when(j > 0)
                def _wait_prev():
                    s_desc((NBUF - 1) % NBUF, NMETA - 1).wait()
            else:
                s_desc((i - 1) % NBUF, i - 1).wait()

            @pl.when(k + NMETA - 1 < NCHUNK)
            def _next_meta():
                m_start(k + NMETA - 1, tm)

            @pl.when(k + GDEPTH < NCHUNK)
            def _next_gather():
                m_wait(k + GDEPTH, tg)
                g_desc(bg, tg).start()

            pltpu.async_copy(rows[b], agg_sh.at[mdst[t]], ssem[b],
                             add=True)
        return carry

    lax.fori_loop(0, GROUPS, group_body, 0)

    for r in range(TAIL):
        k = GROUPS * NMETA + r
        b = k % NBUF
        t = k % NMETA
        g_desc(b, t).wait()
        scale(b, t)
        s_desc((k - 1) % NBUF, (k - 1) % NMETA).wait()
        if k + NMETA - 1 < NCHUNK:
            m_start(k + NMETA - 1, (k + NMETA - 1) % NMETA)
        if k + GDEPTH < NCHUNK:
            m_wait(k + GDEPTH, (k + GDEPTH) % NMETA)
            g_desc((k + GDEPTH) % NBUF, (k + GDEPTH) % NMETA).start()
        pltpu.async_copy(rows[b], agg_sh.at[mdst[t]], ssem[b], add=True)
    s_desc((NCHUNK - 1) % NBUF, (NCHUNK - 1) % NMETA).wait()
    plsc.subcore_barrier()

    @pl.when(s < SPLIT_TILE)
    def _writeout_a():
        pltpu.sync_copy(agg_sh.at[pl.ds(base, ROWS_A)],
                        out_hbm.at[c, pl.ds(base, ROWS_A)])

    @pl.when(s >= SPLIT_TILE)
    def _writeout_b():
        pltpu.sync_copy(agg_sh.at[pl.ds(base, ROWS_B)],
                        out_hbm.at[c, pl.ds(base, ROWS_B)])


_sc_agg = pl.kernel(
    _sc_agg_body,
    out_type=jax.ShapeDtypeStruct((NC, N, D_IN), jnp.float32),
    mesh=plsc.VectorSubcoreMesh(core_axis_name="c", subcore_axis_name="s"),
    compiler_params=pltpu.CompilerParams(needs_layout_passes=False),
    scratch_types=(
        [pltpu.VMEM((CHUNK,), jnp.int32) for _ in range(NMETA)]
        + [pltpu.VMEM((CHUNK,), jnp.int32) for _ in range(NMETA)]
        + [pltpu.VMEM((CHUNK,), jnp.float32) for _ in range(NMETA)]
        + [pltpu.VMEM((CHUNK, D_IN), jnp.float32) for _ in range(NBUF)]
        + [pltpu.VMEM_SHARED((N, D_IN), jnp.float32)]
        + [pltpu.SemaphoreType.DMA for _ in range(NBUF * 2 + NMETA)]
    ),
)


def _mlp_body(a_ref, x_ref, wrel_ref, wroot_ref, wh_ref, wout_ref,
              brel_ref, bh_ref, bout_ref, o_ref):
    agg = (a_ref[0] + a_ref[1]).astype(jnp.bfloat16)
    h = (jnp.dot(agg, wrel_ref[...], preferred_element_type=jnp.float32)
         + jnp.dot(x_ref[...], wroot_ref[...],
                   preferred_element_type=jnp.float32)
         + brel_ref[...])
    h = jnp.maximum(h, 0.0).astype(jnp.bfloat16)
    h2 = jnp.dot(h, wh_ref[...], preferred_element_type=jnp.float32) + bh_ref[...]
    h2 = jnp.maximum(h2, 0.0) + jnp.log1p(jnp.exp(-jnp.abs(h2)))
    o_ref[...] = (jnp.dot(h2.astype(jnp.bfloat16), wout_ref[...],
                          preferred_element_type=jnp.float32)
                  + bout_ref[...])


_BLK = 2000


def _mlp(agg2, x, wrel_t, wroot_t, wh_t, wout_t, brel, bh, bout):
    grid = (N // _BLK,)
    return pl.pallas_call(
        _mlp_body,
        grid=grid,
        in_specs=[
            pl.BlockSpec((NC, _BLK, D_IN), lambda i: (0, i, 0)),
            pl.BlockSpec((_BLK, D_IN), lambda i: (i, 0)),
            pl.BlockSpec((D_IN, D_H), lambda i: (0, 0)),
            pl.BlockSpec((D_IN, D_H), lambda i: (0, 0)),
            pl.BlockSpec((D_H, D_H), lambda i: (0, 0)),
            pl.BlockSpec((D_H, D_OUT), lambda i: (0, 0)),
            pl.BlockSpec((1, D_H), lambda i: (0, 0)),
            pl.BlockSpec((1, D_H), lambda i: (0, 0)),
            pl.BlockSpec((1, D_OUT), lambda i: (0, 0)),
        ],
        out_specs=pl.BlockSpec((_BLK, D_OUT), lambda i: (i, 0)),
        out_shape=jax.ShapeDtypeStruct((N, D_OUT), jnp.float32),
    )(agg2, x, wrel_t, wroot_t, wh_t, wout_t, brel, bh, bout)


def kernel(feature_data, edge_info, edge_weights, W_rel, b_rel, W_root,
           W_h, b_h, W_out, b_out):
    ei = edge_info.astype(jnp.int32)
    agg2 = _sc_agg(feature_data, ei[0], ei[1], edge_weights)
    bf = jnp.bfloat16
    return _mlp(agg2, feature_data.astype(bf), W_rel.T.astype(bf),
                W_root.T.astype(bf), W_h.T.astype(bf), W_out.T.astype(bf),
                b_rel[None, :], b_h[None, :], b_out[None, :])


# speedup vs baseline: 1.0012x; 1.0012x over previous
import jax
import jax.numpy as jnp
from jax import lax
from jax.experimental import pallas as pl
from jax.experimental.pallas import tpu as pltpu
from jax.experimental.pallas import tpu_sc as plsc

N = 10000
E = 320000
D_IN = 128
D_H = 256
D_OUT = 128

NC = 2
NS = 16
NW = NC * NS
EPW = E // NW
CHUNK = 80
NCHUNK = EPW // CHUNK
NBUF = 4
NMETA = 8
GDEPTH = NBUF - 1
GROUPS = NCHUNK // NMETA
TAIL = NCHUNK - GROUPS * NMETA
ROWS_A = 624
ROWS_B = 632
SPLIT_TILE = 14


def _sc_agg_body(x_hbm, src_hbm, dst_hbm, w_hbm, out_hbm, *scr):
    msrc = scr[0:NMETA]
    mdst = scr[NMETA:2 * NMETA]
    mw = scr[2 * NMETA:3 * NMETA]
    rows = scr[3 * NMETA:3 * NMETA + NBUF]
    agg_sh = scr[3 * NMETA + NBUF]
    sems = scr[3 * NMETA + NBUF + 1:]
    gsem = sems[0:NBUF]
    ssem = sems[NBUF:2 * NBUF]
    msem = sems[2 * NBUF:2 * NBUF + NMETA]
    c = lax.axis_index("c")
    s = lax.axis_index("s")
    wid = c * NS + s
    ebase = wid * EPW

    def m_descs(k, t):
        sl = pl.ds(ebase + k * CHUNK, CHUNK)
        return (pltpu.make_async_copy(src_hbm.at[sl], msrc[t], msem[t]),
                pltpu.make_async_copy(dst_hbm.at[sl], mdst[t], msem[t]),
                pltpu.make_async_copy(w_hbm.at[sl], mw[t], msem[t]))

    def m_start(k, t):
        for d in m_descs(k, t):
            d.start()

    def m_wait(k, t):
        for d in m_descs(k, t):
            d.wait()

    def g_desc(b, t):
        return pltpu.make_async_copy(x_hbm.at[msrc[t]], rows[b], gsem[b])

    def s_desc(b, t):
        return pltpu.make_async_copy(rows[b], agg_sh.at[mdst[t]], ssem[b])

    def scale(b, t):
        @plsc.parallel_loop(0, CHUNK, step=1, unroll=8)
        def _scale(e):
            wsp = plsc.load_gather(mw[t], [jnp.full((16,), e, jnp.int32)])
            for jj in range(D_IN // 16):
                sl = pl.ds(jj * 16, 16)
                rows[b][e, sl] = rows[b][e, sl] * wsp

    zbuf = rows[NBUF - 1]
    for t in range(NMETA - 1):
        m_start(t, t)
    for t in range(GDEPTH):
        m_wait(t, t)
        g_desc(t, t).start()

    zv = jnp.zeros((16,), jnp.float32)

    def zero_body(i, carry):
        for j in range(D_IN // 16):
            zbuf[i, pl.ds(j * 16, 16)] = zv
        return carry

    lax.fori_loop(0, CHUNK, zero_body, 0)

    base = jnp.where(s < SPLIT_TILE, s * ROWS_A, s * ROWS_B - 112)

    def _slice_copies(nrows, copy_fn):
        nfull = nrows // CHUNK
        rem = nrows - nfull * CHUNK
        for kk in range(nfull):
            copy_fn(pl.ds(base + kk * CHUNK, CHUNK), CHUNK)
        if rem:
            copy_fn(pl.ds(base + nfull * CHUNK, rem), rem)

    def _zero_copy(dst_sl, nr):
        pltpu.sync_copy(zbuf.at[pl.ds(0, nr)], agg_sh.at[dst_sl])

    @pl.when(s < SPLIT_TILE)
    def _zero_slice_a():
        _slice_copies(ROWS_A, _zero_copy)

    @pl.when(s >= SPLIT_TILE)
    def _zero_slice_b():
        _slice_copies(ROWS_B, _zero_copy)

    plsc.subcore_barrier()

    def group_body(j, carry):
        for i in range(NMETA):
            k = j * NMETA + i
            b = i % NBUF
            t = i
            bg = (i + GDEPTH) % NBUF
            tg = (i + GDEPTH) % NMETA
            tm = (i + NMETA - 1) % NMETA
            g_desc(b, t).wait()
            scale(b, t)

            if i == 0:
                @pl.when(j > 0)
                def _wait_prev():
                    s_desc((NBUF - 1) % NBUF, NMETA - 1).wait()
            else:
                s_desc((i - 1) % NBUF, i - 1).wait()

            @pl.when(k + NMETA - 1 < NCHUNK)
            def _next_meta():
                m_start(k + NMETA - 1, tm)

            @pl.when(k + GDEPTH < NCHUNK)
            def _next_gather():
                m_wait(k + GDEPTH, tg)
                g_desc(bg, tg).start()

            pltpu.async_copy(rows[b], agg_sh.at[mdst[t]], ssem[b],
                             add=True)
        return carry

    lax.fori_loop(0, GROUPS, group_body, 0)

    for r in range(TAIL):
        k = GROUPS * NMETA + r
        b = k % NBUF
        t = k % NMETA
        g_desc(b, t).wait()
        scale(b, t)
        s_desc((k - 1) % NBUF, (k - 1) % NMETA).wait()
        if k + NMETA - 1 < NCHUNK:
            m_start(k + NMETA - 1, (k + NMETA - 1) % NMETA)
        if k + GDEPTH < NCHUNK:
            m_wait(k + GDEPTH, (k + GDEPTH) % NMETA)
            g_desc((k + GDEPTH) % NBUF, (k + GDEPTH) % NMETA).start()
        pltpu.async_copy(rows[b], agg_sh.at[mdst[t]], ssem[b], add=True)
    s_desc((NCHUNK - 1) % NBUF, (NCHUNK - 1) % NMETA).wait()
    plsc.subcore_barrier()

    @pl.when(s < SPLIT_TILE)
    def _writeout_a():
        pltpu.sync_copy(agg_sh.at[pl.ds(base, ROWS_A)],
                        out_hbm.at[c, pl.ds(base, ROWS_A)])

    @pl.when(s >= SPLIT_TILE)
    def _writeout_b():
        pltpu.sync_copy(agg_sh.at[pl.ds(base, ROWS_B)],
                        out_hbm.at[c, pl.ds(base, ROWS_B)])


_sc_agg = pl.kernel(
    _sc_agg_body,
    out_type=jax.ShapeDtypeStruct((NC, N, D_IN), jnp.float32),
    mesh=plsc.VectorSubcoreMesh(core_axis_name="c", subcore_axis_name="s"),
    compiler_params=pltpu.CompilerParams(needs_layout_passes=False),
    scratch_types=(
        [pltpu.VMEM((CHUNK,), jnp.int32) for _ in range(NMETA)]
        + [pltpu.VMEM((CHUNK,), jnp.int32) for _ in range(NMETA)]
        + [pltpu.VMEM((CHUNK,), jnp.float32) for _ in range(NMETA)]
        + [pltpu.VMEM((CHUNK, D_IN), jnp.float32) for _ in range(NBUF)]
        + [pltpu.VMEM_SHARED((N, D_IN), jnp.float32)]
        + [pltpu.SemaphoreType.DMA for _ in range(NBUF * 2 + NMETA)]
    ),
)


def _root_body(x_ref, wroot_ref, brel_ref, r_ref):
    r_ref[...] = (jnp.dot(x_ref[...], wroot_ref[...],
                          preferred_element_type=jnp.float32)
                  + brel_ref[...])


def _mlp_body(a_ref, root_ref, wrel_ref, wh_ref, wout_ref,
              bh_ref, bout_ref, o_ref):
    agg = (a_ref[0] + a_ref[1]).astype(jnp.bfloat16)
    h = (jnp.dot(agg, wrel_ref[...], preferred_element_type=jnp.float32)
         + root_ref[...])
    h = jnp.maximum(h, 0.0).astype(jnp.bfloat16)
    h2 = jnp.dot(h, wh_ref[...], preferred_element_type=jnp.float32) + bh_ref[...]
    h2 = jnp.maximum(h2, 0.0) + jnp.log1p(jnp.exp(-jnp.abs(h2)))
    o_ref[...] = (jnp.dot(h2.astype(jnp.bfloat16), wout_ref[...],
                          preferred_element_type=jnp.float32)
                  + bout_ref[...])


_BLK = 2000


def _root(x, wroot_t, brel):
    grid = (N // _BLK,)
    return pl.pallas_call(
        _root_body,
        grid=grid,
        in_specs=[
            pl.BlockSpec((_BLK, D_IN), lambda i: (i, 0)),
            pl.BlockSpec((D_IN, D_H), lambda i: (0, 0)),
            pl.BlockSpec((1, D_H), lambda i: (0, 0)),
        ],
        out_specs=pl.BlockSpec((_BLK, D_H), lambda i: (i, 0)),
        out_shape=jax.ShapeDtypeStruct((N, D_H), jnp.float32),
    )(x, wroot_t, brel)


def _mlp(agg2, root, wrel_t, wh_t, wout_t, bh, bout):
    grid = (N // _BLK,)
    return pl.pallas_call(
        _mlp_body,
        grid=grid,
        in_specs=[
            pl.BlockSpec((NC, _BLK, D_IN), lambda i: (0, i, 0)),
            pl.BlockSpec((_BLK, D_H), lambda i: (i, 0)),
            pl.BlockSpec((D_IN, D_H), lambda i: (0, 0)),
            pl.BlockSpec((D_H, D_H), lambda i: (0, 0)),
            pl.BlockSpec((D_H, D_OUT), lambda i: (0, 0)),
            pl.BlockSpec((1, D_H), lambda i: (0, 0)),
            pl.BlockSpec((1, D_OUT), lambda i: (0, 0)),
        ],
        out_specs=pl.BlockSpec((_BLK, D_OUT), lambda i: (i, 0)),
        out_shape=jax.ShapeDtypeStruct((N, D_OUT), jnp.float32),
    )(agg2, root, wrel_t, wh_t, wout_t, bh, bout)


def kernel(feature_data, edge_info, edge_weights, W_rel, b_rel, W_root,
           W_h, b_h, W_out, b_out):
    ei = edge_info.astype(jnp.int32)
    bf = jnp.bfloat16
    root = _root(feature_data.astype(bf), W_root.T.astype(bf),
                 b_rel[None, :])
    agg2 = _sc_agg(feature_data, ei[0], ei[1], edge_weights)
    return _mlp(agg2, root, W_rel.T.astype(bf), W_h.T.astype(bf),
                W_out.T.astype(bf), b_h[None, :], b_out[None, :])


# speedup vs baseline: 1.0124x; 1.0112x over previous
import jax
import jax.numpy as jnp
from jax import lax
from jax.experimental import pallas as pl
from jax.experimental.pallas import tpu as pltpu
from jax.experimental.pallas import tpu_sc as plsc

N = 10000
E = 320000
D_IN = 128
D_H = 256
D_OUT = 128

NC = 2
NS = 16
NW = NC * NS
EPW = E // NW
CHUNK = 80
NCHUNK = EPW // CHUNK
NBUF = 4
NMETA = 8
GDEPTH = NBUF - 1
GROUPS = NCHUNK // NMETA
TAIL = NCHUNK - GROUPS * NMETA
ROWS_A = 624
ROWS_B = 632
SPLIT_TILE = 14


def _sc_agg_body(x_hbm, src_hbm, dst_hbm, w_hbm, out_hbm, *scr):
    msrc = scr[0:NMETA]
    mdst = scr[NMETA:2 * NMETA]
    mw = scr[2 * NMETA:3 * NMETA]
    rows = scr[3 * NMETA:3 * NMETA + NBUF]
    agg_sh = scr[3 * NMETA + NBUF]
    sems = scr[3 * NMETA + NBUF + 1:]
    gsem = sems[0:NBUF]
    ssem = sems[NBUF:2 * NBUF]
    msem = sems[2 * NBUF:2 * NBUF + NMETA]
    c = lax.axis_index("c")
    s = lax.axis_index("s")
    wid = c * NS + s
    ebase = wid * EPW

    def m_descs(k, t):
        sl = pl.ds(ebase + k * CHUNK, CHUNK)
        return (pltpu.make_async_copy(src_hbm.at[sl], msrc[t], msem[t]),
                pltpu.make_async_copy(dst_hbm.at[sl], mdst[t], msem[t]),
                pltpu.make_async_copy(w_hbm.at[sl], mw[t], msem[t]))

    def m_start(k, t):
        for d in m_descs(k, t):
            d.start()

    def m_wait(k, t):
        for d in m_descs(k, t):
            d.wait()

    def g_desc(b, t):
        return pltpu.make_async_copy(x_hbm.at[msrc[t]], rows[b], gsem[b])

    def s_desc(b, t):
        return pltpu.make_async_copy(rows[b], agg_sh.at[mdst[t]], ssem[b])

    def scale(b, t):
        @plsc.parallel_loop(0, CHUNK, step=1, unroll=8)
        def _scale(e):
            wsp = plsc.load_gather(mw[t], [jnp.full((16,), e, jnp.int32)])
            for jj in range(D_IN // 16):
                sl = pl.ds(jj * 16, 16)
                rows[b][e, sl] = rows[b][e, sl] * wsp

    zbuf = rows[NBUF - 1]
    for t in range(NMETA - 1):
        m_start(t, t)
    for t in range(GDEPTH):
        m_wait(t, t)
        g_desc(t, t).start()

    zv = jnp.zeros((16,), jnp.float32)

    def zero_body(i, carry):
        for j in range(D_IN // 16):
            zbuf[i, pl.ds(j * 16, 16)] = zv
        return carry

    lax.fori_loop(0, CHUNK, zero_body, 0)

    base = jnp.where(s < SPLIT_TILE, s * ROWS_A, s * ROWS_B - 112)

    def _slice_copies(nrows, copy_fn):
        nfull = nrows // CHUNK
        rem = nrows - nfull * CHUNK
        for kk in range(nfull):
            copy_fn(pl.ds(base + kk * CHUNK, CHUNK), CHUNK)
        if rem:
            copy_fn(pl.ds(base + nfull * CHUNK, rem), rem)

    def _zero_copy(dst_sl, nr):
        pltpu.sync_copy(zbuf.at[pl.ds(0, nr)], agg_sh.at[dst_sl])

    @pl.when(s < SPLIT_TILE)
    def _zero_slice_a():
        _slice_copies(ROWS_A, _zero_copy)

    @pl.when(s >= SPLIT_TILE)
    def _zero_slice_b():
        _slice_copies(ROWS_B, _zero_copy)

    plsc.subcore_barrier()

    def group_body(j, carry):
        for i in range(NMETA):
            k = j * NMETA + i
            b = i % NBUF
            t = i
            bg = (i + GDEPTH) % NBUF
            tg = (i + GDEPTH) % NMETA
            tm = (i + NMETA - 1) % NMETA
            g_desc(b, t).wait()
            scale(b, t)

            if i == 0:
                @pl.when(j > 0)
                def _wait_prev():
                    s_desc((NBUF - 1) % NBUF, NMETA - 1).wait()
            else:
                s_desc((i - 1) % NBUF, i - 1).wait()

            @pl.when(k + NMETA - 1 < NCHUNK)
            def _next_meta():
                m_start(k + NMETA - 1, tm)

            @pl.when(k + GDEPTH < NCHUNK)
            def _next_gather():
                m_wait(k + GDEPTH, tg)
                g_desc(bg, tg).start()

            pltpu.async_copy(rows[b], agg_sh.at[mdst[t]], ssem[b],
                             add=True)
        return carry

    lax.fori_loop(0, GROUPS, group_body, 0)

    for r in range(TAIL):
        k = GROUPS * NMETA + r
        b = k % NBUF
        t = k % NMETA
        g_desc(b, t).wait()
        scale(b, t)
        s_desc((k - 1) % NBUF, (k - 1) % NMETA).wait()
        if k + NMETA - 1 < NCHUNK:
            m_start(k + NMETA - 1, (k + NMETA - 1) % NMETA)
        if k + GDEPTH < NCHUNK:
            m_wait(k + GDEPTH, (k + GDEPTH) % NMETA)
            g_desc((k + GDEPTH) % NBUF, (k + GDEPTH) % NMETA).start()
        pltpu.async_copy(rows[b], agg_sh.at[mdst[t]], ssem[b], add=True)
    s_desc((NCHUNK - 1) % NBUF, (NCHUNK - 1) % NMETA).wait()
    plsc.subcore_barrier()

    @pl.when(s < SPLIT_TILE)
    def _writeout_a():
        pltpu.sync_copy(agg_sh.at[pl.ds(base, ROWS_A)],
                        out_hbm.at[c, pl.ds(base, ROWS_A)])

    @pl.when(s >= SPLIT_TILE)
    def _writeout_b():
        pltpu.sync_copy(agg_sh.at[pl.ds(base, ROWS_B)],
                        out_hbm.at[c, pl.ds(base, ROWS_B)])


_sc_agg = pl.kernel(
    _sc_agg_body,
    out_type=jax.ShapeDtypeStruct((NC, N, D_IN), jnp.float32),
    mesh=plsc.VectorSubcoreMesh(core_axis_name="c", subcore_axis_name="s"),
    compiler_params=pltpu.CompilerParams(needs_layout_passes=False),
    scratch_types=(
        [pltpu.VMEM((CHUNK,), jnp.int32) for _ in range(NMETA)]
        + [pltpu.VMEM((CHUNK,), jnp.int32) for _ in range(NMETA)]
        + [pltpu.VMEM((CHUNK,), jnp.float32) for _ in range(NMETA)]
        + [pltpu.VMEM((CHUNK, D_IN), jnp.float32) for _ in range(NBUF)]
        + [pltpu.VMEM_SHARED((N, D_IN), jnp.float32)]
        + [pltpu.SemaphoreType.DMA for _ in range(NBUF * 2 + NMETA)]
    ),
)


def _mlp_body(a_ref, x_ref, wrel_ref, wroot_ref, wh_ref, wout_ref,
              brel_ref, bh_ref, bout_ref, o_ref):
    agg = (a_ref[0] + a_ref[1]).astype(jnp.bfloat16)
    h = (jnp.dot(agg, wrel_ref[...], preferred_element_type=jnp.float32)
         + jnp.dot(x_ref[...], wroot_ref[...],
                   preferred_element_type=jnp.float32)
         + brel_ref[...])
    h = jnp.maximum(h, 0.0).astype(jnp.bfloat16)
    h2 = jnp.dot(h, wh_ref[...], preferred_element_type=jnp.float32) + bh_ref[...]
    h2 = jnp.maximum(h2, 0.0) + jnp.log1p(jnp.exp(-jnp.abs(h2)))
    o_ref[...] = (jnp.dot(h2.astype(jnp.bfloat16), wout_ref[...],
                          preferred_element_type=jnp.float32)
                  + bout_ref[...])


_BLK = 2000


def _mlp(agg2, x, wrel_t, wroot_t, wh_t, wout_t, brel, bh, bout):
    grid = (N // _BLK,)
    return pl.pallas_call(
        _mlp_body,
        grid=grid,
        in_specs=[
            pl.BlockSpec((NC, _BLK, D_IN), lambda i: (0, i, 0)),
            pl.BlockSpec((_BLK, D_IN), lambda i: (i, 0)),
            pl.BlockSpec((D_IN, D_H), lambda i: (0, 0)),
            pl.BlockSpec((D_IN, D_H), lambda i: (0, 0)),
            pl.BlockSpec((D_H, D_H), lambda i: (0, 0)),
            pl.BlockSpec((D_H, D_OUT), lambda i: (0, 0)),
            pl.BlockSpec((1, D_H), lambda i: (0, 0)),
            pl.BlockSpec((1, D_H), lambda i: (0, 0)),
            pl.BlockSpec((1, D_OUT), lambda i: (0, 0)),
        ],
        out_specs=pl.BlockSpec((_BLK, D_OUT), lambda i: (i, 0)),
        out_shape=jax.ShapeDtypeStruct((N, D_OUT), jnp.float32),
    )(agg2, x, wrel_t, wroot_t, wh_t, wout_t, brel, bh, bout)


def kernel(feature_data, edge_info, edge_weights, W_rel, b_rel, W_root,
           W_h, b_h, W_out, b_out):
    ei = edge_info.astype(jnp.int32)
    agg2 = _sc_agg(feature_data, ei[0], ei[1], edge_weights)
    bf = jnp.bfloat16
    return _mlp(agg2, feature_data.astype(bf), W_rel.T.astype(bf),
                W_root.T.astype(bf), W_h.T.astype(bf), W_out.T.astype(bf),
                b_rel[None, :], b_h[None, :], b_out[None, :])
